# Initial kernel scaffold; baseline (speedup 1.0000x reference)
import jax
import jax.numpy as jnp
from jax.experimental import pallas as pl


def kernel(x, Wg, W1, b1, W2, b2, ln_g, ln_b, ln1_g, ln1_b, ln2_g, ln2_b, ln3_g, ln3_b, ln4_g, ln4_b):
    raise NotImplementedError("write your pallas kernel here")


# speedup vs baseline: 2.0927x; 2.0927x over previous
import jax
import jax.numpy as jnp
from jax import lax
from jax.experimental import pallas as pl

F32 = jnp.float32
E = 24
CAP = 43
SPAD = 48
NCH = 4
TOK = 512
DIN = 768
DHID = 2048
ROWS = E * SPAD
HI = lax.Precision.HIGHEST


def _routing_body(ei_ref, gv_ref, codes_ref, gates_ref):
    ei = ei_ref[0]
    gv = gv_ref[0]
    i0 = ei[:, 0:1]
    i1 = ei[:, 1:2]
    g0 = gv[:, 0:1]
    g1 = gv[:, 1:2]
    idx = lax.broadcasted_iota(jnp.int32, (TOK, E), 1)
    oh0 = (idx == i0).astype(F32)
    oh1 = (idx == i1).astype(F32)
    oh = jnp.concatenate([oh0, oh1], axis=0)
    ri = lax.broadcasted_iota(jnp.int32, (2 * TOK, 2 * TOK), 0)
    ci = lax.broadcasted_iota(jnp.int32, (2 * TOK, 2 * TOK), 1)
    ls = (ci < ri).astype(F32)
    csum = jnp.dot(ls, oh)
    pos = jnp.sum(csum * oh, axis=-1, keepdims=True).astype(jnp.int32)
    keep = pos < CAP
    slot = jnp.where(keep, pos, CAP)
    e_flat = jnp.concatenate([i0, i1], axis=0)
    code = e_flat * SPAD + slot
    g_flat = jnp.concatenate([g0, g1], axis=0)
    g_flat = jnp.where(keep, g_flat, 0.0)
    codes_ref[0] = code
    gates_ref[0] = g_flat


def _dispatch_body(codes_ref, x_ref, buf_ref, *, d):
    code = codes_ref[0]
    c0 = code[:TOK]
    c1 = code[TOK:]
    ri = lax.broadcasted_iota(jnp.int32, (TOK, ROWS), 1)
    mt = ((ri == c0) | (ri == c1)).astype(F32)
    xc = x_ref[0]
    bufc = lax.dot_general(mt, xc, (((0,), (0,)), ((), ())), precision=HI)
    buf_ref[:, 0] = bufc.reshape(E, SPAD, d)


def _mm_body(buf_ref, w_ref, b_ref, y_ref, *, d_in, d_out, prec):
    bufe = buf_ref[0].reshape(NCH * SPAD, d_in)
    y = jnp.dot(bufe, w_ref[0], precision=prec) + b_ref[0]
    y_ref[0] = y.reshape(NCH, SPAD, d_out)


def _combine1_body(codes_ref, gates_ref, y_ref, g_ref, b_ref, h_ref):
    code = codes_ref[0]
    gate = gates_ref[0]
    c0, c1 = code[:TOK], code[TOK:]
    g0, g1 = gate[:TOK], gate[TOK:]
    ri = lax.broadcasted_iota(jnp.int32, (TOK, ROWS), 1)
    cm = (ri == c0).astype(F32) * g0 + (ri == c1).astype(F32) * g1
    y = y_ref[:, 0].reshape(ROWS, DHID)
    comb = jnp.dot(cm, y, precision=HI)
    mu = jnp.mean(comb, axis=-1, keepdims=True)
    var = jnp.mean((comb - mu) ** 2, axis=-1, keepdims=True)
    h = (comb - mu) * lax.rsqrt(var + 1e-6) * g_ref[0] + b_ref[0]
    h_ref[0] = jnp.maximum(h, 0.0)


def _combine2_body(codes_ref, gates_ref, y_ref, x_ref, g_ref, b_ref, o_ref):
    code = codes_ref[0]
    gate = gates_ref[0]
    c0, c1 = code[:TOK], code[TOK:]
    g0, g1 = gate[:TOK], gate[TOK:]
    ri = lax.broadcasted_iota(jnp.int32, (TOK, ROWS), 1)
    cm = (ri == c0).astype(F32) * g0 + (ri == c1).astype(F32) * g1
    y = y_ref[:, 0].reshape(ROWS, DIN)
    comb = jnp.dot(cm, y, precision=HI) + x_ref[0]
    mu = jnp.mean(comb, axis=-1, keepdims=True)
    var = jnp.mean((comb - mu) ** 2, axis=-1, keepdims=True)
    o_ref[0] = (comb - mu) * lax.rsqrt(var + 1e-6) * g_ref[...] + b_ref[...]


def kernel(x, Wg, W1, b1, W2, b2, ln_g, ln_b, ln1_g, ln1_b, ln2_g, ln2_b,
           ln3_g, ln3_b, ln4_g, ln4_b):
    x4 = x.reshape(NCH, TOK, DIN)

    b, s, c = x.shape
    xr = x.reshape(b, NCH, TOK, c)
    eis, gvs = [], []
    for i in range(NCH):
        xc = xr[:, i].reshape(TOK, c)
        logits = xc @ Wg
        probs = jax.nn.softmax(logits, axis=-1)
        topv, topi = jax.lax.top_k(probs, 2)
        topv = topv / (jnp.sum(topv, axis=-1, keepdims=True) + 1e-9)
        eis.append(topi)
        gvs.append(topv)
    ei = jnp.stack(eis)
    gv = jnp.stack(gvs)

    codes, gates = pl.pallas_call(
        _routing_body,
        grid=(NCH,),
        in_specs=[
            pl.BlockSpec((1, TOK, 2), lambda c: (c, 0, 0)),
            pl.BlockSpec((1, TOK, 2), lambda c: (c, 0, 0)),
        ],
        out_specs=[
            pl.BlockSpec((1, 2 * TOK, 1), lambda c: (c, 0, 0)),
            pl.BlockSpec((1, 2 * TOK, 1), lambda c: (c, 0, 0)),
        ],
        out_shape=[
            jax.ShapeDtypeStruct((NCH, 2 * TOK, 1), jnp.int32),
            jax.ShapeDtypeStruct((NCH, 2 * TOK, 1), F32),
        ],
    )(ei, gv)

    def dispatch(src, d):
        return pl.pallas_call(
            lambda c_ref, s_ref, o_ref: _dispatch_body(c_ref, s_ref, o_ref, d=d),
            grid=(NCH,),
            in_specs=[
                pl.BlockSpec((1, 2 * TOK, 1), lambda c: (c, 0, 0)),
                pl.BlockSpec((1, TOK, d), lambda c: (c, 0, 0)),
            ],
            out_specs=pl.BlockSpec((E, 1, SPAD, d), lambda c: (0, c, 0, 0)),
            out_shape=jax.ShapeDtypeStruct((E, NCH, SPAD, d), F32),
        )(codes, src)

    def expert_mm(buf, W, b, d_in, d_out):
        return pl.pallas_call(
            lambda b_ref, w_ref, bb_ref, y_ref: _mm_body(
                b_ref, w_ref, bb_ref, y_ref, d_in=d_in, d_out=d_out, prec=HI),
            grid=(E,),
            in_specs=[
                pl.BlockSpec((1, NCH, SPAD, d_in), lambda e: (e, 0, 0, 0)),
                pl.BlockSpec((1, d_in, d_out), lambda e: (e, 0, 0)),
                pl.BlockSpec((1, 1, d_out), lambda e: (e, 0, 0)),
            ],
            out_specs=pl.BlockSpec((1, NCH, SPAD, d_out), lambda e: (e, 0, 0, 0)),
            out_shape=jax.ShapeDtypeStruct((E, NCH, SPAD, d_out), F32),
        )(buf, W, b.reshape(E, 1, d_out))

    buf1 = dispatch(x4, DIN)
    y1 = expert_mm(buf1, W1, b1, DIN, DHID)

    lng = jnp.stack([ln1_g, ln2_g, ln3_g, ln4_g]).reshape(NCH, 1, DHID)
    lnb = jnp.stack([ln1_b, ln2_b, ln3_b, ln4_b]).reshape(NCH, 1, DHID)
    h = pl.pallas_call(
        _combine1_body,
        grid=(NCH,),
        in_specs=[
            pl.BlockSpec((1, 2 * TOK, 1), lambda c: (c, 0, 0)),
            pl.BlockSpec((1, 2 * TOK, 1), lambda c: (c, 0, 0)),
            pl.BlockSpec((E, 1, SPAD, DHID), lambda c: (0, c, 0, 0)),
            pl.BlockSpec((1, 1, DHID), lambda c: (c, 0, 0)),
            pl.BlockSpec((1, 1, DHID), lambda c: (c, 0, 0)),
        ],
        out_specs=pl.BlockSpec((1, TOK, DHID), lambda c: (c, 0, 0)),
        out_shape=jax.ShapeDtypeStruct((NCH, TOK, DHID), F32),
    )(codes, gates, y1, lng, lnb)

    buf2 = dispatch(h, DHID)
    y2 = expert_mm(buf2, W2, b2, DHID, DIN)

    out = pl.pallas_call(
        _combine2_body,
        grid=(NCH,),
        in_specs=[
            pl.BlockSpec((1, 2 * TOK, 1), lambda c: (c, 0, 0)),
            pl.BlockSpec((1, 2 * TOK, 1), lambda c: (c, 0, 0)),
            pl.BlockSpec((E, 1, SPAD, DIN), lambda c: (0, c, 0, 0)),
            pl.BlockSpec((1, TOK, DIN), lambda c: (c, 0, 0)),
            pl.BlockSpec((1, DIN), lambda c: (0, 0)),
            pl.BlockSpec((1, DIN), lambda c: (0, 0)),
        ],
        out_specs=pl.BlockSpec((1, TOK, DIN), lambda c: (c, 0, 0)),
        out_shape=jax.ShapeDtypeStruct((NCH, TOK, DIN), F32),
    )(codes, gates, y2, x4, ln_g.reshape(1, DIN), ln_b.reshape(1, DIN))

    return out.reshape(1, NCH * TOK, DIN)


# speedup vs baseline: 2.8817x; 1.3770x over previous
import jax
import jax.numpy as jnp
from jax import lax
from jax.experimental import pallas as pl

F32 = jnp.float32
E = 24
CAP = 43
SPAD = 48
NCH = 4
TOK = 512
DIN = 768
DHID = 2048
ROWS = E * SPAD
HI = lax.Precision.HIGHEST


def _routing_body(ei_ref, gv_ref, codes_ref, gates_ref):
    ei = ei_ref[0]
    gv = gv_ref[0]
    i0 = ei[:, 0:1]
    i1 = ei[:, 1:2]
    g0 = gv[:, 0:1]
    g1 = gv[:, 1:2]
    idx = lax.broadcasted_iota(jnp.int32, (TOK, E), 1)
    oh0 = (idx == i0).astype(F32)
    oh1 = (idx == i1).astype(F32)
    oh = jnp.concatenate([oh0, oh1], axis=0)
    ri = lax.broadcasted_iota(jnp.int32, (2 * TOK, 2 * TOK), 0)
    ci = lax.broadcasted_iota(jnp.int32, (2 * TOK, 2 * TOK), 1)
    ls = (ci < ri).astype(F32)
    csum = jnp.dot(ls, oh)
    pos = jnp.sum(csum * oh, axis=-1, keepdims=True).astype(jnp.int32)
    keep = pos < CAP
    slot = jnp.where(keep, pos, CAP)
    e_flat = jnp.concatenate([i0, i1], axis=0)
    code = e_flat * SPAD + slot
    g_flat = jnp.concatenate([g0, g1], axis=0)
    g_flat = jnp.where(keep, g_flat, 0.0)
    codes_ref[0] = code
    gates_ref[0] = g_flat


BF16 = jnp.bfloat16


def _dispatch_body(codes_ref, x_ref, buf_ref, *, d):
    code = codes_ref[0]
    c0 = code[:TOK]
    c1 = code[TOK:]
    ri = lax.broadcasted_iota(jnp.int32, (TOK, ROWS), 1)
    mt = ((ri == c0) | (ri == c1)).astype(BF16)
    xc = x_ref[0].astype(BF16)
    bufc = lax.dot_general(mt, xc, (((0,), (0,)), ((), ())),
                           preferred_element_type=F32)
    buf_ref[:, 0] = bufc.reshape(E, SPAD, d)


def _mm_body(buf_ref, w_ref, b_ref, y_ref, *, d_in, d_out):
    bufe = buf_ref[0].reshape(NCH * SPAD, d_in).astype(BF16)
    y = jnp.dot(bufe, w_ref[0], preferred_element_type=F32) + b_ref[0]
    y_ref[0] = y.reshape(NCH, SPAD, d_out)


def _combine1_body(codes_ref, gates_ref, y_ref, g_ref, b_ref, h_ref):
    code = codes_ref[0]
    gate = gates_ref[0]
    c0, c1 = code[:TOK], code[TOK:]
    g0, g1 = gate[:TOK], gate[TOK:]
    ri = lax.broadcasted_iota(jnp.int32, (TOK, ROWS), 1)
    cm = ((ri == c0).astype(F32) * g0 + (ri == c1).astype(F32) * g1).astype(BF16)
    y = y_ref[:, 0].reshape(ROWS, DHID).astype(BF16)
    comb = jnp.dot(cm, y, preferred_element_type=F32)
    mu = jnp.mean(comb, axis=-1, keepdims=True)
    var = jnp.mean((comb - mu) ** 2, axis=-1, keepdims=True)
    h = (comb - mu) * lax.rsqrt(var + 1e-6) * g_ref[0] + b_ref[0]
    h_ref[0] = jnp.maximum(h, 0.0)


def _combine2_body(codes_ref, gates_ref, y_ref, x_ref, g_ref, b_ref, o_ref):
    code = codes_ref[0]
    gate = gates_ref[0]
    c0, c1 = code[:TOK], code[TOK:]
    g0, g1 = gate[:TOK], gate[TOK:]
    ri = lax.broadcasted_iota(jnp.int32, (TOK, ROWS), 1)
    cm = ((ri == c0).astype(F32) * g0 + (ri == c1).astype(F32) * g1).astype(BF16)
    y = y_ref[:, 0].reshape(ROWS, DIN).astype(BF16)
    comb = jnp.dot(cm, y, preferred_element_type=F32) + x_ref[0]
    mu = jnp.mean(comb, axis=-1, keepdims=True)
    var = jnp.mean((comb - mu) ** 2, axis=-1, keepdims=True)
    o_ref[0] = (comb - mu) * lax.rsqrt(var + 1e-6) * g_ref[...] + b_ref[...]


def kernel(x, Wg, W1, b1, W2, b2, ln_g, ln_b, ln1_g, ln1_b, ln2_g, ln2_b,
           ln3_g, ln3_b, ln4_g, ln4_b):
    x4 = x.reshape(NCH, TOK, DIN)

    b, s, c = x.shape
    xr = x.reshape(b, NCH, TOK, c)
    eis, gvs = [], []
    for i in range(NCH):
        xc = xr[:, i].reshape(TOK, c)
        logits = xc @ Wg
        probs = jax.nn.softmax(logits, axis=-1)
        topv, topi = jax.lax.top_k(probs, 2)
        topv = topv / (jnp.sum(topv, axis=-1, keepdims=True) + 1e-9)
        eis.append(topi)
        gvs.append(topv)
    ei = jnp.stack(eis)
    gv = jnp.stack(gvs)

    codes, gates = pl.pallas_call(
        _routing_body,
        grid=(NCH,),
        in_specs=[
            pl.BlockSpec((1, TOK, 2), lambda c: (c, 0, 0)),
            pl.BlockSpec((1, TOK, 2), lambda c: (c, 0, 0)),
        ],
        out_specs=[
            pl.BlockSpec((1, 2 * TOK, 1), lambda c: (c, 0, 0)),
            pl.BlockSpec((1, 2 * TOK, 1), lambda c: (c, 0, 0)),
        ],
        out_shape=[
            jax.ShapeDtypeStruct((NCH, 2 * TOK, 1), jnp.int32),
            jax.ShapeDtypeStruct((NCH, 2 * TOK, 1), F32),
        ],
    )(ei, gv)

    def dispatch(src, d):
        return pl.pallas_call(
            lambda c_ref, s_ref, o_ref: _dispatch_body(c_ref, s_ref, o_ref, d=d),
            grid=(NCH,),
            in_specs=[
                pl.BlockSpec((1, 2 * TOK, 1), lambda c: (c, 0, 0)),
                pl.BlockSpec((1, TOK, d), lambda c: (c, 0, 0)),
            ],
            out_specs=pl.BlockSpec((E, 1, SPAD, d), lambda c: (0, c, 0, 0)),
            out_shape=jax.ShapeDtypeStruct((E, NCH, SPAD, d), F32),
        )(codes, src)

    def expert_mm(buf, W, b, d_in, d_out):
        return pl.pallas_call(
            lambda b_ref, w_ref, bb_ref, y_ref: _mm_body(
                b_ref, w_ref, bb_ref, y_ref, d_in=d_in, d_out=d_out),
            grid=(E,),
            in_specs=[
                pl.BlockSpec((1, NCH, SPAD, d_in), lambda e: (e, 0, 0, 0)),
                pl.BlockSpec((1, d_in, d_out), lambda e: (e, 0, 0)),
                pl.BlockSpec((1, 1, d_out), lambda e: (e, 0, 0)),
            ],
            out_specs=pl.BlockSpec((1, NCH, SPAD, d_out), lambda e: (e, 0, 0, 0)),
            out_shape=jax.ShapeDtypeStruct((E, NCH, SPAD, d_out), F32),
        )(buf, W.astype(BF16), b.reshape(E, 1, d_out))

    buf1 = dispatch(x4, DIN)
    y1 = expert_mm(buf1, W1, b1, DIN, DHID)

    lng = jnp.stack([ln1_g, ln2_g, ln3_g, ln4_g]).reshape(NCH, 1, DHID)
    lnb = jnp.stack([ln1_b, ln2_b, ln3_b, ln4_b]).reshape(NCH, 1, DHID)
    h = pl.pallas_call(
        _combine1_body,
        grid=(NCH,),
        in_specs=[
            pl.BlockSpec((1, 2 * TOK, 1), lambda c: (c, 0, 0)),
            pl.BlockSpec((1, 2 * TOK, 1), lambda c: (c, 0, 0)),
            pl.BlockSpec((E, 1, SPAD, DHID), lambda c: (0, c, 0, 0)),
            pl.BlockSpec((1, 1, DHID), lambda c: (c, 0, 0)),
            pl.BlockSpec((1, 1, DHID), lambda c: (c, 0, 0)),
        ],
        out_specs=pl.BlockSpec((1, TOK, DHID), lambda c: (c, 0, 0)),
        out_shape=jax.ShapeDtypeStruct((NCH, TOK, DHID), F32),
    )(codes, gates, y1, lng, lnb)

    buf2 = dispatch(h, DHID)
    y2 = expert_mm(buf2, W2, b2, DHID, DIN)

    out = pl.pallas_call(
        _combine2_body,
        grid=(NCH,),
        in_specs=[
            pl.BlockSpec((1, 2 * TOK, 1), lambda c: (c, 0, 0)),
            pl.BlockSpec((1, 2 * TOK, 1), lambda c: (c, 0, 0)),
            pl.BlockSpec((E, 1, SPAD, DIN), lambda c: (0, c, 0, 0)),
            pl.BlockSpec((1, TOK, DIN), lambda c: (c, 0, 0)),
            pl.BlockSpec((1, DIN), lambda c: (0, 0)),
            pl.BlockSpec((1, DIN), lambda c: (0, 0)),
        ],
        out_specs=pl.BlockSpec((1, TOK, DIN), lambda c: (c, 0, 0)),
        out_shape=jax.ShapeDtypeStruct((NCH, TOK, DIN), F32),
    )(codes, gates, y2, x4, ln_g.reshape(1, DIN), ln_b.reshape(1, DIN))

    return out.reshape(1, NCH * TOK, DIN)


# speedup vs baseline: 5.0444x; 1.7505x over previous
import jax
import jax.numpy as jnp
from jax import lax
from jax.experimental import pallas as pl

F32 = jnp.float32
BF16 = jnp.bfloat16
E = 24
CAP = 43
SPAD = 48
NCH = 4
TOK = 512
DIN = 768
DHID = 2048
ROWS = E * SPAD


def _routing_body(ei_ref, gv_ref, codes_ref, gates_ref):
    ei = ei_ref[0]
    gv = gv_ref[0]
    i0 = ei[:, 0:1]
    i1 = ei[:, 1:2]
    g0 = gv[:, 0:1]
    g1 = gv[:, 1:2]
    idx = lax.broadcasted_iota(jnp.int32, (TOK, E), 1)
    oh0 = (idx == i0).astype(BF16)
    oh1 = (idx == i1).astype(BF16)
    oh = jnp.concatenate([oh0, oh1], axis=0)
    ri = lax.broadcasted_iota(jnp.int32, (2 * TOK, 2 * TOK), 0)
    ci = lax.broadcasted_iota(jnp.int32, (2 * TOK, 2 * TOK), 1)
    ls = (ci < ri).astype(BF16)
    csum = jnp.dot(ls, oh, preferred_element_type=F32)
    pos = jnp.sum(csum * oh.astype(F32), axis=-1, keepdims=True).astype(jnp.int32)
    keep = pos < CAP
    slot = jnp.where(keep, pos, CAP)
    e_flat = jnp.concatenate([i0, i1], axis=0)
    code = e_flat * SPAD + slot
    g_flat = jnp.concatenate([g0, g1], axis=0)
    g_flat = jnp.where(keep, g_flat, 0.0)
    codes_ref[0] = code
    gates_ref[0] = g_flat


def _dispatch_body(codes_ref, x_ref, buf_ref, *, d):
    code = codes_ref[0]
    c0 = code[:TOK]
    c1 = code[TOK:]
    ri = lax.broadcasted_iota(jnp.int32, (TOK, ROWS), 1)
    mt = ((ri == c0) | (ri == c1)).astype(BF16)
    xc = x_ref[0].astype(BF16)
    bufc = lax.dot_general(mt, xc, (((0,), (0,)), ((), ())),
                           preferred_element_type=F32)
    buf_ref[:, 0] = bufc.astype(BF16).reshape(E, SPAD, d)


def _mm_body(buf_ref, w_ref, b_ref, y_ref, *, d_in, d_out):
    bufe = buf_ref[0].reshape(NCH * SPAD, d_in)
    w = w_ref[0].astype(BF16)
    y = jnp.dot(bufe, w, preferred_element_type=F32) + b_ref[0]
    y_ref[0] = y.astype(BF16).reshape(NCH, SPAD, d_out)


def _comb1_disp2_body(codes_ref, gates_ref, y_ref, g_ref, b_ref, buf_ref):
    code = codes_ref[0]
    gate = gates_ref[0]
    c0, c1 = code[:TOK], code[TOK:]
    g0, g1 = gate[:TOK], gate[TOK:]
    ri = lax.broadcasted_iota(jnp.int32, (TOK, ROWS), 1)
    cm = ((ri == c0).astype(F32) * g0 + (ri == c1).astype(F32) * g1).astype(BF16)
    y = y_ref[:, 0].reshape(ROWS, DHID)
    comb = jnp.dot(cm, y, preferred_element_type=F32)
    mu = jnp.mean(comb, axis=-1, keepdims=True)
    var = jnp.mean((comb - mu) ** 2, axis=-1, keepdims=True)
    h = (comb - mu) * lax.rsqrt(var + 1e-6) * g_ref[0] + b_ref[0]
    h = jnp.maximum(h, 0.0).astype(BF16)
    mt = ((ri == c0) | (ri == c1)).astype(BF16)
    buf2 = lax.dot_general(mt, h, (((0,), (0,)), ((), ())),
                           preferred_element_type=F32)
    buf_ref[:, 0] = buf2.astype(BF16).reshape(E, SPAD, DHID)


def _combine2_body(codes_ref, gates_ref, y_ref, x_ref, g_ref, b_ref, o_ref):
    code = codes_ref[0]
    gate = gates_ref[0]
    c0, c1 = code[:TOK], code[TOK:]
    g0, g1 = gate[:TOK], gate[TOK:]
    ri = lax.broadcasted_iota(jnp.int32, (TOK, ROWS), 1)
    cm = ((ri == c0).astype(F32) * g0 + (ri == c1).astype(F32) * g1).astype(BF16)
    y = y_ref[:, 0].reshape(ROWS, DIN)
    comb = jnp.dot(cm, y, preferred_element_type=F32) + x_ref[0]
    mu = jnp.mean(comb, axis=-1, keepdims=True)
    var = jnp.mean((comb - mu) ** 2, axis=-1, keepdims=True)
    o_ref[0] = (comb - mu) * lax.rsqrt(var + 1e-6) * g_ref[...] + b_ref[...]


def kernel(x, Wg, W1, b1, W2, b2, ln_g, ln_b, ln1_g, ln1_b, ln2_g, ln2_b,
           ln3_g, ln3_b, ln4_g, ln4_b):
    x4 = x.reshape(NCH, TOK, DIN)

    b, s, c = x.shape
    xr = x.reshape(b, NCH, TOK, c)
    eis, gvs = [], []
    for i in range(NCH):
        xc = xr[:, i].reshape(TOK, c)
        logits = xc @ Wg
        probs = jax.nn.softmax(logits, axis=-1)
        topv, topi = jax.lax.top_k(probs, 2)
        topv = topv / (jnp.sum(topv, axis=-1, keepdims=True) + 1e-9)
        eis.append(topi)
        gvs.append(topv)
    ei = jnp.stack(eis)
    gv = jnp.stack(gvs)

    codes, gates = pl.pallas_call(
        _routing_body,
        grid=(NCH,),
        in_specs=[
            pl.BlockSpec((1, TOK, 2), lambda c: (c, 0, 0)),
            pl.BlockSpec((1, TOK, 2), lambda c: (c, 0, 0)),
        ],
        out_specs=[
            pl.BlockSpec((1, 2 * TOK, 1), lambda c: (c, 0, 0)),
            pl.BlockSpec((1, 2 * TOK, 1), lambda c: (c, 0, 0)),
        ],
        out_shape=[
            jax.ShapeDtypeStruct((NCH, 2 * TOK, 1), jnp.int32),
            jax.ShapeDtypeStruct((NCH, 2 * TOK, 1), F32),
        ],
    )(ei, gv)

    buf1 = pl.pallas_call(
        lambda c_ref, s_ref, o_ref: _dispatch_body(c_ref, s_ref, o_ref, d=DIN),
        grid=(NCH,),
        in_specs=[
            pl.BlockSpec((1, 2 * TOK, 1), lambda c: (c, 0, 0)),
            pl.BlockSpec((1, TOK, DIN), lambda c: (c, 0, 0)),
        ],
        out_specs=pl.BlockSpec((E, 1, SPAD, DIN), lambda c: (0, c, 0, 0)),
        out_shape=jax.ShapeDtypeStruct((E, NCH, SPAD, DIN), BF16),
    )(codes, x4)

    def expert_mm(buf, W, b, d_in, d_out):
        return pl.pallas_call(
            lambda b_ref, w_ref, bb_ref, y_ref: _mm_body(
                b_ref, w_ref, bb_ref, y_ref, d_in=d_in, d_out=d_out),
            grid=(E,),
            in_specs=[
                pl.BlockSpec((1, NCH, SPAD, d_in), lambda e: (e, 0, 0, 0)),
                pl.BlockSpec((1, d_in, d_out), lambda e: (e, 0, 0)),
                pl.BlockSpec((1, 1, d_out), lambda e: (e, 0, 0)),
            ],
            out_specs=pl.BlockSpec((1, NCH, SPAD, d_out), lambda e: (e, 0, 0, 0)),
            out_shape=jax.ShapeDtypeStruct((E, NCH, SPAD, d_out), BF16),
        )(buf, W, b.reshape(E, 1, d_out))

    y1 = expert_mm(buf1, W1, b1, DIN, DHID)

    lng = jnp.stack([ln1_g, ln2_g, ln3_g, ln4_g]).reshape(NCH, 1, DHID)
    lnb = jnp.stack([ln1_b, ln2_b, ln3_b, ln4_b]).reshape(NCH, 1, DHID)
    buf2 = pl.pallas_call(
        _comb1_disp2_body,
        grid=(NCH,),
        in_specs=[
            pl.BlockSpec((1, 2 * TOK, 1), lambda c: (c, 0, 0)),
            pl.BlockSpec((1, 2 * TOK, 1), lambda c: (c, 0, 0)),
            pl.BlockSpec((E, 1, SPAD, DHID), lambda c: (0, c, 0, 0)),
            pl.BlockSpec((1, 1, DHID), lambda c: (c, 0, 0)),
            pl.BlockSpec((1, 1, DHID), lambda c: (c, 0, 0)),
        ],
        out_specs=pl.BlockSpec((E, 1, SPAD, DHID), lambda c: (0, c, 0, 0)),
        out_shape=jax.ShapeDtypeStruct((E, NCH, SPAD, DHID), BF16),
    )(codes, gates, y1, lng, lnb)

    y2 = expert_mm(buf2, W2, b2, DHID, DIN)

    out = pl.pallas_call(
        _combine2_body,
        grid=(NCH,),
        in_specs=[
            pl.BlockSpec((1, 2 * TOK, 1), lambda c: (c, 0, 0)),
            pl.BlockSpec((1, 2 * TOK, 1), lambda c: (c, 0, 0)),
            pl.BlockSpec((E, 1, SPAD, DIN), lambda c: (0, c, 0, 0)),
            pl.BlockSpec((1, TOK, DIN), lambda c: (c, 0, 0)),
            pl.BlockSpec((1, DIN), lambda c: (0, 0)),
            pl.BlockSpec((1, DIN), lambda c: (0, 0)),
        ],
        out_specs=pl.BlockSpec((1, TOK, DIN), lambda c: (c, 0, 0)),
        out_shape=jax.ShapeDtypeStruct((NCH, TOK, DIN), F32),
    )(codes, gates, y2, x4, ln_g.reshape(1, DIN), ln_b.reshape(1, DIN))

    return out.reshape(1, NCH * TOK, DIN)
